# baseline (speedup 1.0000x reference)
import functools

import jax
import jax.numpy as jnp
from jax import lax
from jax.experimental import pallas as pl
from jax.experimental.pallas import tpu as pltpu
from jax.experimental.pallas import tpu_sc as plsc

NC = 2
NS = 16
NW = NC * NS
WIN = 128
CH = 6
F = 16

_f32 = jnp.float32
_i32 = jnp.int32


def _seg_sum_body(np_rows, rchunks, z_hbm, src_hbm, dst_hbm, out_hbm,
                  src_buf, dst_buf, zrow, rows, acc, sem_i, sem_g, sem_s):
    c = lax.axis_index("c")
    s = lax.axis_index("s")
    wid = s * NC + c
    rows_per_tile = np_rows // NS
    tile_base = s * rows_per_tile

    for i in range(WIN):
        zrow[i, :] = jnp.zeros((F,), _f32)

    def _zero(j, carry):
        pltpu.sync_copy(zrow, acc.at[pl.ds(tile_base + j * WIN, WIN)])
        return carry

    lax.fori_loop(0, rows_per_tile // WIN, _zero, 0)
    plsc.subcore_barrier()

    ngroups = rchunks
    edge_base = wid * ngroups * CH

    def _stage_idx(group, b):
        sl = pl.ds(edge_base + group * CH, CH)
        pltpu.async_copy(src_hbm.at[sl], src_buf.at[b], sem_i)
        pltpu.async_copy(dst_hbm.at[sl], dst_buf.at[b], sem_i)

    def _wait_idx(b):
        pltpu.make_async_copy(src_hbm.at[pl.ds(0, CH)], src_buf.at[b], sem_i).wait()
        pltpu.make_async_copy(dst_hbm.at[pl.ds(0, CH)], dst_buf.at[b], sem_i).wait()

    def _fire_gathers(b):
        for r in range(CH):
            pltpu.async_copy(z_hbm.at[src_buf.at[b, r]], rows.at[b, r], sem_g)

    def _drain_gathers(b):
        for r in range(CH):
            pltpu.make_async_copy(z_hbm.at[pl.ds(0, WIN)], rows.at[b, r], sem_g).wait()

    def _fire_scatters(b):
        for r in range(CH):
            pltpu.async_copy(rows.at[b, r], acc.at[dst_buf.at[b, r]], sem_s, add=True)

    def _drain_scatters(b):
        for r in range(CH):
            pltpu.make_async_copy(rows.at[b, r], acc.at[pl.ds(0, WIN)], sem_s).wait()

    for j in range(WIN // 16):
        dst_buf[0, 0, pl.ds(j * 16, 16)] = lax.iota(_i32, 16) + j * 16
    for r in range(CH):
        pltpu.async_copy(zrow, acc.at[dst_buf.at[0, 0]], sem_s, add=True)
    _stage_idx(0, 0)
    _wait_idx(0)
    _stage_idx(1, 1)
    _fire_gathers(0)

    def _half(b, k2, goff):
        g_next = 2 * k2 + goff + 1
        _drain_gathers(b)
        _drain_scatters(1 - b)
        _fire_scatters(b)
        _wait_idx(1 - b)
        _stage_idx(jnp.minimum(g_next + 1, ngroups - 1), b)
        _fire_gathers(1 - b)

    def _pair(k2, carry):
        _half(0, k2, 0)
        _half(1, k2, 1)
        return carry

    lax.fori_loop(0, ngroups // 2, _pair, 0)

    _drain_gathers(0)
    _drain_scatters(1)
    _wait_idx(0)
    plsc.subcore_barrier()

    def _flush(j, carry):
        base = tile_base + j * WIN
        pltpu.sync_copy(acc.at[pl.ds(base, WIN)], zrow)
        pltpu.sync_copy(zrow, out_hbm.at[pl.ds(c * np_rows + base, WIN)])
        return carry

    lax.fori_loop(0, rows_per_tile // WIN, _flush, 0)


def _make_seg_sum(np_rows, rchunks):
    mesh = plsc.VectorSubcoreMesh(core_axis_name="c", subcore_axis_name="s",
                                  num_cores=NC, num_subcores=NS)
    return pl.kernel(
        functools.partial(_seg_sum_body, np_rows, rchunks),
        out_type=jax.ShapeDtypeStruct((NC * np_rows, F), _f32),
        mesh=mesh,
        scratch_types=[
            pltpu.VMEM((2, CH, WIN), _i32),
            pltpu.VMEM((2, CH, WIN), _i32),
            pltpu.VMEM((WIN, F), _f32),
            pltpu.VMEM((2, CH, WIN, F), _f32),
            pltpu.VMEM_SHARED((np_rows, F), _f32),
            pltpu.SemaphoreType.DMA,
            pltpu.SemaphoreType.DMA,
            pltpu.SemaphoreType.DMA,
        ],
        compiler_params=pltpu.CompilerParams(use_tc_tiling_on_sc=False),
    )


def _dense1_kern(a0, a1, x, wr, wo, b, o):
    s0 = a0[...] + a1[...]
    o[...] = jnp.maximum(s0 * wr[...] + x[...] * wo[...] + b[...], 0.0)


def _dense2_kern(a0, a1, z1, wr, wo, b, o):
    agg = a0[...] + a1[...]
    o[...] = (jnp.dot(agg, wr[...], preferred_element_type=_f32) + b[...]
              + jnp.dot(z1[...], wo[...], preferred_element_type=_f32))


def _dense3_kern(a0, a1, z2, wr, wo, b, o):
    agg = a0[...] + a1[...]
    y = (jnp.dot(agg, wr[...], preferred_element_type=_f32) + b[...]
         + jnp.dot(z2[...], wo[...], preferred_element_type=_f32))
    o[...] = jnp.maximum(y, 0.0)


def kernel(x, edge_index, W_rel1, b1, W_root1, W_rel2, b2, W_root2,
           W_rel3, b3, W_root3):
    n = x.shape[0]
    e = edge_index.shape[1]

    np_rows = -(-(n + 1) // (NS * WIN)) * (NS * WIN)
    np8 = np_rows // 8

    rchunks = -(-e // (NW * WIN * CH))
    rchunks += rchunks % 2
    ep = NW * rchunks * CH * WIN
    pad = ep - e
    src = edge_index[0].astype(_i32)
    dst = edge_index[1].astype(_i32)
    fill = jnp.full((pad,), n, dtype=_i32)
    src = jnp.concatenate([src, fill]).reshape(NW * rchunks * CH, WIN)
    dst = jnp.concatenate([dst, fill]).reshape(NW * rchunks * CH, WIN)

    x_pad = jnp.pad(x, ((0, np_rows - n), (0, 0)))
    x16_128 = jnp.repeat(x_pad[:, 0], F).reshape(np8, 128)

    seg = _make_seg_sum(np_rows, rchunks)

    eye8 = jnp.eye(8, dtype=_f32)
    wr2 = jnp.kron(eye8, W_rel2)
    wo2 = jnp.kron(eye8, W_root2)
    wr3 = jnp.kron(eye8, jnp.tile(W_rel3, (1, F)))
    wo3 = jnp.kron(eye8, jnp.tile(W_root3, (1, F)))
    wr1 = jnp.tile(W_rel1.reshape(1, F), (1, 8))
    wo1 = jnp.tile(W_root1.reshape(1, F), (1, 8))
    b1t = jnp.tile(b1.reshape(1, F), (1, 8))
    b2t = jnp.tile(b2.reshape(1, F), (1, 8))
    b3t = jnp.tile(b3.reshape(1, 1), (1, 128))

    for blk in (1600, 1280, 1024, 800, 640, 512, 400, 320, 256):
        if np8 % blk == 0:
            break
    nb = np8 // blk
    grid = (nb,)
    row_spec = pl.BlockSpec((blk, 128), lambda i: (i, 0))
    part_specs = [
        pl.BlockSpec((blk, 128), lambda i: (i, 0)),
        pl.BlockSpec((blk, 128), lambda i, _nb=nb: (i + _nb, 0)),
    ]
    w_spec = pl.BlockSpec((128, 128), lambda i: (0, 0))
    wrow_spec = pl.BlockSpec((1, 128), lambda i: (0, 0))

    p0 = seg(x16_128.reshape(np_rows, F), src, dst).reshape(2 * np8, 128)
    z1_128 = pl.pallas_call(
        _dense1_kern, grid=grid,
        in_specs=part_specs + [row_spec, wrow_spec, wrow_spec, wrow_spec],
        out_specs=row_spec,
        out_shape=jax.ShapeDtypeStruct((np8, 128), _f32),
    )(p0, p0, x16_128, wr1, wo1, b1t)

    p1 = seg(z1_128.reshape(np_rows, F), src, dst).reshape(2 * np8, 128)
    z2_128 = pl.pallas_call(
        _dense2_kern, grid=grid,
        in_specs=part_specs + [row_spec, w_spec, w_spec, wrow_spec],
        out_specs=row_spec,
        out_shape=jax.ShapeDtypeStruct((np8, 128), _f32),
    )(p1, p1, z1_128, wr2, wo2, b2t)

    p2 = seg(z2_128.reshape(np_rows, F), src, dst).reshape(2 * np8, 128)
    y128 = pl.pallas_call(
        _dense3_kern, grid=grid,
        in_specs=part_specs + [row_spec, w_spec, w_spec, wrow_spec],
        out_specs=row_spec,
        out_shape=jax.ShapeDtypeStruct((np8, 128), _f32),
    )(p2, p2, z2_128, wr3, wo3, b3t)

    return y128.reshape(np_rows, F)[:n, :1]


# speedup vs baseline: 1.5782x; 1.5782x over previous
import functools

import jax
import jax.numpy as jnp
from jax import lax
from jax.experimental import pallas as pl
from jax.experimental.pallas import tpu as pltpu
from jax.experimental.pallas import tpu_sc as plsc

NC = 2
NS = 16
NW = NC * NS
WIN = 128
CH = 4
F = 16

_f32 = jnp.float32
_i32 = jnp.int32


def _seg_sum_body(np_rows, rchunks, z_hbm, src_hbm, dst_hbm, out_hbm,
                  src_buf, dst_buf, dst_s, zrow, rows, acc, sem_i, sem_g, sem_s):
    c = lax.axis_index("c")
    s = lax.axis_index("s")
    wid = s * NC + c
    rows_per_tile = np_rows // NS
    tile_base = s * rows_per_tile

    for i in range(WIN):
        zrow[i, :] = jnp.zeros((F,), _f32)

    def _zero(j, carry):
        pltpu.sync_copy(zrow, acc.at[pl.ds(tile_base + j * WIN, WIN)])
        return carry

    lax.fori_loop(0, rows_per_tile // WIN, _zero, 0)
    plsc.subcore_barrier()

    ngroups = rchunks
    edge_base = wid * ngroups * CH

    def _stage_idx(group, b):
        sl = pl.ds(edge_base + group * CH, CH)
        pltpu.async_copy(src_hbm.at[sl], src_buf.at[b], sem_i)
        pltpu.async_copy(dst_hbm.at[sl], dst_buf.at[b], sem_i)

    def _wait_idx(b):
        pltpu.make_async_copy(src_hbm.at[pl.ds(0, CH)], src_buf.at[b], sem_i).wait()
        pltpu.make_async_copy(dst_hbm.at[pl.ds(0, CH)], dst_buf.at[b], sem_i).wait()

    def _fire_gathers(b):
        for r in range(CH):
            pltpu.async_copy(z_hbm.at[src_buf.at[b, r]], rows.at[b, r], sem_g)

    def _drain_gathers(b):
        for r in range(CH):
            pltpu.make_async_copy(z_hbm.at[pl.ds(0, WIN)], rows.at[b, r], sem_g).wait()

    def _fire_scatters(b):
        for r in range(CH):
            for j in range(WIN // 16):
                dst_s[b, r, pl.ds(j * 16, 16)] = dst_buf[b, r, pl.ds(j * 16, 16)]
        for r in range(CH):
            pltpu.async_copy(rows.at[b, r], acc.at[dst_s.at[b, r]], sem_s, add=True)

    def _drain_scatters(b):
        for r in range(CH):
            pltpu.make_async_copy(rows.at[b, r], acc.at[pl.ds(0, WIN)], sem_s).wait()

    for j in range(WIN // 16):
        dst_s[0, 0, pl.ds(j * 16, 16)] = lax.iota(_i32, 16) + j * 16
    for r in range(CH):
        pltpu.async_copy(zrow, acc.at[dst_s.at[0, 0]], sem_s, add=True)
    _stage_idx(0, 0)
    _wait_idx(0)
    _stage_idx(1, 1)
    _fire_gathers(0)

    def _half(b, k2, goff):
        g_next = 2 * k2 + goff + 1
        _drain_gathers(b)
        _drain_scatters(1 - b)
        _fire_scatters(b)
        _wait_idx(1 - b)
        _stage_idx(jnp.minimum(g_next + 1, ngroups - 1), b)
        _fire_gathers(1 - b)

    def _pair(k2, carry):
        _half(0, k2, 0)
        _half(1, k2, 1)
        return carry

    lax.fori_loop(0, ngroups // 2, _pair, 0)

    _drain_gathers(0)
    _drain_scatters(1)
    _wait_idx(0)
    plsc.subcore_barrier()

    def _flush(j, carry):
        base = tile_base + j * WIN
        pltpu.sync_copy(acc.at[pl.ds(base, WIN)], zrow)
        pltpu.sync_copy(zrow, out_hbm.at[pl.ds(c * np_rows + base, WIN)])
        return carry

    lax.fori_loop(0, rows_per_tile // WIN, _flush, 0)


def _make_seg_sum(np_rows, rchunks):
    mesh = plsc.VectorSubcoreMesh(core_axis_name="c", subcore_axis_name="s",
                                  num_cores=NC, num_subcores=NS)
    return pl.kernel(
        functools.partial(_seg_sum_body, np_rows, rchunks),
        out_type=jax.ShapeDtypeStruct((NC * np_rows, F), _f32),
        mesh=mesh,
        scratch_types=[
            pltpu.VMEM((2, CH, WIN), _i32),
            pltpu.VMEM((2, CH, WIN), _i32),
            pltpu.VMEM((2, CH, WIN), _i32),
            pltpu.VMEM((WIN, F), _f32),
            pltpu.VMEM((2, CH, WIN, F), _f32),
            pltpu.VMEM_SHARED((np_rows, F), _f32),
            pltpu.SemaphoreType.DMA,
            pltpu.SemaphoreType.DMA,
            pltpu.SemaphoreType.DMA,
        ],
        compiler_params=pltpu.CompilerParams(use_tc_tiling_on_sc=False),
    )


def _dense1_kern(a0, a1, x, wr, wo, b, o):
    s0 = a0[...] + a1[...]
    o[...] = jnp.maximum(s0 * wr[...] + x[...] * wo[...] + b[...], 0.0)


def _dense2_kern(a0, a1, z1, wr, wo, b, o):
    agg = a0[...] + a1[...]
    o[...] = (jnp.dot(agg, wr[...], preferred_element_type=_f32) + b[...]
              + jnp.dot(z1[...], wo[...], preferred_element_type=_f32))


def _dense3_kern(a0, a1, z2, wr, wo, b, o):
    agg = a0[...] + a1[...]
    y = (jnp.dot(agg, wr[...], preferred_element_type=_f32) + b[...]
         + jnp.dot(z2[...], wo[...], preferred_element_type=_f32))
    o[...] = jnp.maximum(y, 0.0)


def kernel(x, edge_index, W_rel1, b1, W_root1, W_rel2, b2, W_root2,
           W_rel3, b3, W_root3):
    n = x.shape[0]
    e = edge_index.shape[1]

    np_rows = -(-(n + 1025) // (NS * WIN)) * (NS * WIN)
    np8 = np_rows // 8

    rchunks = -(-e // (NW * WIN * CH))
    rchunks += rchunks % 2
    ep = NW * rchunks * CH * WIN
    pad = ep - e
    src = edge_index[0].astype(_i32)
    dst = edge_index[1].astype(_i32)
    fill = n + (jnp.arange(pad, dtype=_i32) % 1024)
    src = jnp.concatenate([src, fill]).reshape(NW * rchunks * CH, WIN)
    dst = jnp.concatenate([dst, fill]).reshape(NW * rchunks * CH, WIN)

    x_pad = jnp.pad(x, ((0, np_rows - n), (0, 0)))
    x16_128 = jnp.repeat(x_pad[:, 0], F).reshape(np8, 128)

    seg = _make_seg_sum(np_rows, rchunks)

    eye8 = jnp.eye(8, dtype=_f32)
    wr2 = jnp.kron(eye8, W_rel2)
    wo2 = jnp.kron(eye8, W_root2)
    wr3 = jnp.kron(eye8, jnp.tile(W_rel3, (1, F)))
    wo3 = jnp.kron(eye8, jnp.tile(W_root3, (1, F)))
    wr1 = jnp.tile(W_rel1.reshape(1, F), (1, 8))
    wo1 = jnp.tile(W_root1.reshape(1, F), (1, 8))
    b1t = jnp.tile(b1.reshape(1, F), (1, 8))
    b2t = jnp.tile(b2.reshape(1, F), (1, 8))
    b3t = jnp.tile(b3.reshape(1, 1), (1, 128))

    for blk in (1600, 1280, 1024, 800, 640, 512, 400, 320, 256):
        if np8 % blk == 0:
            break
    nb = np8 // blk
    grid = (nb,)
    row_spec = pl.BlockSpec((blk, 128), lambda i: (i, 0))
    part_specs = [
        pl.BlockSpec((blk, 128), lambda i: (i, 0)),
        pl.BlockSpec((blk, 128), lambda i, _nb=nb: (i + _nb, 0)),
    ]
    w_spec = pl.BlockSpec((128, 128), lambda i: (0, 0))
    wrow_spec = pl.BlockSpec((1, 128), lambda i: (0, 0))

    p0 = seg(x16_128.reshape(np_rows, F), src, dst).reshape(2 * np8, 128)
    z1_128 = pl.pallas_call(
        _dense1_kern, grid=grid,
        in_specs=part_specs + [row_spec, wrow_spec, wrow_spec, wrow_spec],
        out_specs=row_spec,
        out_shape=jax.ShapeDtypeStruct((np8, 128), _f32),
    )(p0, p0, x16_128, wr1, wo1, b1t)

    p1 = seg(z1_128.reshape(np_rows, F), src, dst).reshape(2 * np8, 128)
    z2_128 = pl.pallas_call(
        _dense2_kern, grid=grid,
        in_specs=part_specs + [row_spec, w_spec, w_spec, wrow_spec],
        out_specs=row_spec,
        out_shape=jax.ShapeDtypeStruct((np8, 128), _f32),
    )(p1, p1, z1_128, wr2, wo2, b2t)

    p2 = seg(z2_128.reshape(np_rows, F), src, dst).reshape(2 * np8, 128)
    y128 = pl.pallas_call(
        _dense3_kern, grid=grid,
        in_specs=part_specs + [row_spec, w_spec, w_spec, wrow_spec],
        out_specs=row_spec,
        out_shape=jax.ShapeDtypeStruct((np8, 128), _f32),
    )(p2, p2, z2_128, wr3, wo3, b3t)

    return y128.reshape(np_rows, F)[:n, :1]


# speedup vs baseline: 1.9571x; 1.2401x over previous
import functools

import jax
import jax.numpy as jnp
from jax import lax
from jax.experimental import pallas as pl
from jax.experimental.pallas import tpu as pltpu
from jax.experimental.pallas import tpu_sc as plsc

NC = 2
NS = 16
NW = NC * NS
WIN = 128
CH = 4
F = 16

_f32 = jnp.float32
_i32 = jnp.int32


def _seg_sum_body(np_rows, rchunks, z_hbm, src_hbm, dst_hbm, out_hbm,
                  src_buf, dst_buf, dst_s, zrow, rows, acc, sem_i, sem_g, sem_s):
    c = lax.axis_index("c")
    s = lax.axis_index("s")
    wid = s * NC + c
    rows_per_tile = np_rows // NS
    tile_base = s * rows_per_tile

    for i in range(WIN):
        zrow[i, :] = jnp.zeros((F,), _f32)

    def _zero(j, carry):
        pltpu.sync_copy(zrow, acc.at[pl.ds(tile_base + j * WIN, WIN)])
        return carry

    lax.fori_loop(0, rows_per_tile // WIN, _zero, 0)
    plsc.subcore_barrier()

    ngroups = rchunks
    edge_base = wid * ngroups * CH

    def _stage_idx(group, b):
        sl = pl.ds(edge_base + group * CH, CH)
        pltpu.async_copy(src_hbm.at[sl], src_buf.at[b], sem_i)
        pltpu.async_copy(dst_hbm.at[sl], dst_buf.at[b], sem_i)

    def _wait_idx(b):
        pltpu.make_async_copy(src_hbm.at[pl.ds(0, CH)], src_buf.at[b], sem_i).wait()
        pltpu.make_async_copy(dst_hbm.at[pl.ds(0, CH)], dst_buf.at[b], sem_i).wait()

    def _fire_gathers(b):
        for r in range(CH):
            pltpu.async_copy(z_hbm.at[src_buf.at[b, r]], rows.at[b, r], sem_g)

    def _drain_gathers(b):
        for r in range(CH):
            pltpu.make_async_copy(z_hbm.at[pl.ds(0, WIN)], rows.at[b, r], sem_g).wait()

    def _fire_scatters(b):
        for r in range(CH):
            for j in range(WIN // 16):
                dst_s[b, r, pl.ds(j * 16, 16)] = dst_buf[b, r, pl.ds(j * 16, 16)]
        for r in range(CH):
            pltpu.async_copy(rows.at[b, r], acc.at[dst_s.at[b, r]], sem_s, add=True)

    def _drain_scatters(b):
        for r in range(CH):
            pltpu.make_async_copy(rows.at[b, r], acc.at[pl.ds(0, WIN)], sem_s).wait()

    for j in range(WIN // 16):
        dst_s[0, 0, pl.ds(j * 16, 16)] = lax.iota(_i32, 16) + j * 16
    for r in range(CH):
        pltpu.async_copy(zrow, acc.at[dst_s.at[0, 0]], sem_s, add=True)
    _stage_idx(0, 0)
    _wait_idx(0)
    _stage_idx(1, 1)
    _fire_gathers(0)

    def _half(b, k2, goff):
        g_next = 2 * k2 + goff + 1
        _drain_gathers(b)
        _drain_scatters(1 - b)
        _fire_scatters(b)
        _wait_idx(1 - b)
        _stage_idx(jnp.minimum(g_next + 1, ngroups - 1), b)
        _fire_gathers(1 - b)

    def _pair(k2, carry):
        _half(0, k2, 0)
        _half(1, k2, 1)
        return carry

    lax.fori_loop(0, ngroups // 2, _pair, 0)

    _drain_gathers(0)
    _drain_scatters(1)
    _wait_idx(0)
    plsc.subcore_barrier()

    def _flush(j, carry):
        base = tile_base + j * WIN
        pltpu.sync_copy(acc.at[pl.ds(base, WIN)], zrow)
        pltpu.sync_copy(zrow, out_hbm.at[pl.ds(c * np_rows + base, WIN)])
        return carry

    lax.fori_loop(0, rows_per_tile // WIN, _flush, 0)


def _make_seg_sum(np_rows, rchunks):
    mesh = plsc.VectorSubcoreMesh(core_axis_name="c", subcore_axis_name="s",
                                  num_cores=NC, num_subcores=NS)
    return pl.kernel(
        functools.partial(_seg_sum_body, np_rows, rchunks),
        out_type=jax.ShapeDtypeStruct((NC * np_rows, F), _f32),
        mesh=mesh,
        scratch_types=[
            pltpu.VMEM((2, CH, WIN), _i32),
            pltpu.VMEM((2, CH, WIN), _i32),
            pltpu.VMEM((2, CH, WIN), _i32),
            pltpu.VMEM((WIN, F), _f32),
            pltpu.VMEM((2, CH, WIN, F), _f32),
            pltpu.VMEM_SHARED((np_rows, F), _f32),
            pltpu.SemaphoreType.DMA,
            pltpu.SemaphoreType.DMA,
            pltpu.SemaphoreType.DMA,
        ],
        compiler_params=pltpu.CompilerParams(use_tc_tiling_on_sc=False),
    )


def _p0_body(np_rows, rchunks, x_hbm, src_hbm, dst_hbm, out_hbm,
             xv, src_buf, dst_i, vals, zbuf, acc, sem_i, sem_s):
    c = lax.axis_index("c")
    s = lax.axis_index("s")
    wid = s * NC + c
    per_tile = np_rows // NS
    tile_base = s * per_tile

    pltpu.sync_copy(x_hbm, xv)

    def _zinit(j, carry):
        zbuf[pl.ds(j * 16, 16)] = jnp.zeros((16,), _f32)
        return carry

    lax.fori_loop(0, per_tile // 16, _zinit, 0)
    pltpu.sync_copy(zbuf, acc.at[pl.ds(tile_base, per_tile)])
    plsc.subcore_barrier()

    ngroups = rchunks
    cw = CH * WIN
    edge_base = wid * ngroups * cw
    row_base = wid * ngroups * CH

    def _stage_idx(g, bs, ds3):
        pltpu.async_copy(src_hbm.at[pl.ds(edge_base + g * cw, cw)],
                         src_buf.at[pl.ds(bs * cw, cw)], sem_i)
        pltpu.async_copy(dst_hbm.at[pl.ds(row_base + g * CH, CH)],
                         dst_i.at[pl.ds(ds3 * CH, CH)], sem_i)

    def _wait_idx():
        pltpu.make_async_copy(src_hbm.at[pl.ds(0, cw)],
                              src_buf.at[pl.ds(0, cw)], sem_i).wait()
        pltpu.make_async_copy(dst_hbm.at[pl.ds(0, CH)],
                              dst_i.at[pl.ds(0, CH)], sem_i).wait()

    def _drain_scatters():
        for r in range(CH):
            pltpu.make_async_copy(vals.at[pl.ds(0, WIN)],
                                  acc.at[pl.ds(0, WIN)], sem_s).wait()

    _stage_idx(0, 0, 0)

    def _half(b, g):
        _wait_idx()
        _stage_idx(jnp.minimum(g + 1, ngroups - 1), 1 - b, lax.rem(g + 1, 3))
        for r in range(CH):
            for j in range(WIN // 16):
                off = (b * CH + r) * WIN + j * 16
                idx = src_buf[pl.ds(off, 16)]
                vals[pl.ds(off, 16)] = plsc.load_gather(xv, [idx])

        @pl.when(g >= 1)
        def _():
            _drain_scatters()

        dslot = lax.rem(g, 3)
        for r in range(CH):
            pltpu.async_copy(vals.at[pl.ds((b * CH + r) * WIN, WIN)],
                             acc.at[dst_i.at[dslot * CH + r]], sem_s, add=True)

    def _pair(k2, carry):
        _half(0, 2 * k2)
        _half(1, 2 * k2 + 1)
        return carry

    lax.fori_loop(0, ngroups // 2, _pair, 0)
    _drain_scatters()
    _wait_idx()
    plsc.subcore_barrier()

    pltpu.sync_copy(acc.at[pl.ds(tile_base, per_tile)], zbuf)
    pltpu.sync_copy(zbuf, out_hbm.at[pl.ds(c * np_rows + tile_base, per_tile)])


def _make_p0(np_rows, rchunks):
    mesh = plsc.VectorSubcoreMesh(core_axis_name="c", subcore_axis_name="s",
                                  num_cores=NC, num_subcores=NS)
    return pl.kernel(
        functools.partial(_p0_body, np_rows, rchunks),
        out_type=jax.ShapeDtypeStruct((NC * np_rows,), _f32),
        mesh=mesh,
        scratch_types=[
            pltpu.VMEM((np_rows,), _f32),
            pltpu.VMEM((2 * CH * WIN,), _i32),
            pltpu.VMEM((3 * CH, WIN), _i32),
            pltpu.VMEM((2 * CH * WIN,), _f32),
            pltpu.VMEM((np_rows // NS,), _f32),
            pltpu.VMEM_SHARED((np_rows,), _f32),
            pltpu.SemaphoreType.DMA,
            pltpu.SemaphoreType.DMA,
        ],
        compiler_params=pltpu.CompilerParams(use_tc_tiling_on_sc=False,
                                             needs_layout_passes=False),
    )


def _dense1_kern(a0, a1, x, wr, wo, b, o):
    s0 = a0[...] + a1[...]
    o[...] = jnp.maximum(
        jnp.dot(s0, wr[...], preferred_element_type=_f32)
        + jnp.dot(x[...], wo[...], preferred_element_type=_f32) + b[...], 0.0)


def _dense2_kern(a0, a1, z1, wr, wo, b, o):
    agg = a0[...] + a1[...]
    o[...] = (jnp.dot(agg, wr[...], preferred_element_type=_f32) + b[...]
              + jnp.dot(z1[...], wo[...], preferred_element_type=_f32))


def _dense3_kern(a0, a1, z2, wr, wo, b, o):
    agg = a0[...] + a1[...]
    y = (jnp.dot(agg, wr[...], preferred_element_type=_f32) + b[...]
         + jnp.dot(z2[...], wo[...], preferred_element_type=_f32))
    o[...] = jnp.maximum(y, 0.0)


def kernel(x, edge_index, W_rel1, b1, W_root1, W_rel2, b2, W_root2,
           W_rel3, b3, W_root3):
    n = x.shape[0]
    e = edge_index.shape[1]

    np_rows = -(-(n + 1025) // (NS * WIN)) * (NS * WIN)
    np8 = np_rows // 8

    rchunks = -(-e // (NW * WIN * CH))
    rchunks += rchunks % 2
    ep = NW * rchunks * CH * WIN
    pad = ep - e
    src = edge_index[0].astype(_i32)
    dst = edge_index[1].astype(_i32)
    fill = n + (jnp.arange(pad, dtype=_i32) % 1024)
    src = jnp.concatenate([src, fill]).reshape(NW * rchunks * CH, WIN)
    dst = jnp.concatenate([dst, fill]).reshape(NW * rchunks * CH, WIN)

    x_flat = jnp.pad(x[:, 0], (0, np_rows - n))

    seg = _make_seg_sum(np_rows, rchunks)
    p0k = _make_p0(np_rows, rchunks)

    eye8 = jnp.eye(8, dtype=_f32)
    wr2 = jnp.kron(eye8, W_rel2)
    wo2 = jnp.kron(eye8, W_root2)
    wr3 = jnp.kron(eye8, jnp.tile(W_rel3, (1, F)))
    wo3 = jnp.kron(eye8, jnp.tile(W_root3, (1, F)))
    wr1 = jnp.kron(eye8, W_rel1.reshape(1, F))
    wo1 = jnp.kron(eye8, W_root1.reshape(1, F))
    b1t = jnp.tile(b1.reshape(1, F), (1, 8))
    b2t = jnp.tile(b2.reshape(1, F), (1, 8))
    b3t = jnp.tile(b3.reshape(1, 1), (1, 128))

    for blk in (1600, 1280, 1024, 800, 640, 512, 400, 320, 256):
        if np8 % blk == 0:
            break
    nb = np8 // blk
    grid = (nb,)
    row_spec = pl.BlockSpec((blk, 128), lambda i: (i, 0))
    part_specs = [
        pl.BlockSpec((blk, 128), lambda i: (i, 0)),
        pl.BlockSpec((blk, 128), lambda i, _nb=nb: (i + _nb, 0)),
    ]
    w_spec = pl.BlockSpec((128, 128), lambda i: (0, 0))
    wrow_spec = pl.BlockSpec((1, 128), lambda i: (0, 0))

    p0 = p0k(x_flat, src.reshape(-1), dst).reshape(2 * np8, 8)
    n8_specs = [
        pl.BlockSpec((blk, 8), lambda i: (i, 0)),
        pl.BlockSpec((blk, 8), lambda i, _nb=nb: (i + _nb, 0)),
        pl.BlockSpec((blk, 8), lambda i: (i, 0)),
    ]
    w8_spec = pl.BlockSpec((8, 128), lambda i: (0, 0))
    z1_128 = pl.pallas_call(
        _dense1_kern, grid=grid,
        in_specs=n8_specs + [w8_spec, w8_spec, wrow_spec],
        out_specs=row_spec,
        out_shape=jax.ShapeDtypeStruct((np8, 128), _f32),
    )(p0, p0, x_flat.reshape(np8, 8), wr1, wo1, b1t)

    p1 = seg(z1_128.reshape(np_rows, F), src, dst).reshape(2 * np8, 128)
    z2_128 = pl.pallas_call(
        _dense2_kern, grid=grid,
        in_specs=part_specs + [row_spec, w_spec, w_spec, wrow_spec],
        out_specs=row_spec,
        out_shape=jax.ShapeDtypeStruct((np8, 128), _f32),
    )(p1, p1, z1_128, wr2, wo2, b2t)

    p2 = seg(z2_128.reshape(np_rows, F), src, dst).reshape(2 * np8, 128)
    y128 = pl.pallas_call(
        _dense3_kern, grid=grid,
        in_specs=part_specs + [row_spec, w_spec, w_spec, wrow_spec],
        out_specs=row_spec,
        out_shape=jax.ShapeDtypeStruct((np8, 128), _f32),
    )(p2, p2, z2_128, wr3, wo3, b3t)

    return y128.reshape(np_rows, F)[:n, :1]
